# baseline (speedup 1.0000x reference)
import functools

import jax
import jax.numpy as jnp
from jax import lax
from jax.experimental import pallas as pl
from jax.experimental.pallas import tpu as pltpu
from jax.experimental.pallas import tpu_sc as plsc


def _replicate_sc(emb_flat, batch, rows_per_dma, num_workers, nc):
    flat = emb_flat.shape[0]
    chunk = rows_per_dma * flat
    n_chunks = batch // rows_per_dma
    chunks_per_w = n_chunks // num_workers
    mesh = plsc.VectorSubcoreMesh(core_axis_name="c", subcore_axis_name="s")

    @functools.partial(
        pl.kernel,
        mesh=mesh,
        out_type=jax.ShapeDtypeStruct((n_chunks, chunk), jnp.float32),
        scratch_types=[
            pltpu.VMEM((chunk,), jnp.float32),
            pltpu.SemaphoreType.DMA,
        ],
    )
    def replicate(emb_hbm, out_hbm, buf, sem):
        wid = lax.axis_index("s") * nc + lax.axis_index("c")
        for r in range(rows_per_dma):
            pltpu.sync_copy(emb_hbm, buf.at[pl.ds(r * flat, flat)])
        base = wid * chunks_per_w
        handles = [
            pltpu.async_copy(buf, out_hbm.at[base + c], sem)
            for c in range(chunks_per_w)
        ]
        for h in handles:
            h.wait()

    return replicate(emb_flat)


def kernel(item_seqs, emb):
    batch, seq_len = item_seqs.shape
    hidden = emb.shape[1]
    emb_flat = emb[:seq_len].reshape(-1)

    info = plsc.get_sparse_core_info()
    nc, ns = info.num_cores, info.num_subcores
    num_workers = nc * ns

    rows_per_dma = 4
    assert batch % (rows_per_dma * num_workers) == 0

    out = _replicate_sc(emb_flat, batch, rows_per_dma, num_workers, nc)
    return out.reshape(batch, seq_len, hidden)


# speedup vs baseline: 3.8993x; 3.8993x over previous
import jax
import jax.numpy as jnp
from jax.experimental import pallas as pl


def kernel(item_seqs, emb):
    batch, seq_len = item_seqs.shape
    hidden = emb.shape[1]
    bb = 64

    def body(emb_ref, out_ref):
        out_ref[...] = jnp.broadcast_to(
            emb_ref[...][None], (bb, seq_len, hidden)
        )

    out = pl.pallas_call(
        body,
        grid=(batch // bb,),
        in_specs=[pl.BlockSpec((seq_len, hidden), lambda i: (0, 0))],
        out_specs=pl.BlockSpec((bb, seq_len, hidden), lambda i: (i, 0, 0)),
        out_shape=jax.ShapeDtypeStruct((batch, seq_len, hidden), jnp.float32),
    )(emb[:seq_len])
    return out
